# baseline (speedup 1.0000x reference)
import functools

import jax
import jax.numpy as jnp
from jax.experimental import pallas as pl

_DEPTH = 1000
_SPLIT = 512
_CBLK = 64


def _body(c_base, idxt_ref, out_ref):
    i = pl.program_id(0)
    idxt = idxt_ref[...]
    s, n = idxt.shape
    c = jax.lax.broadcasted_iota(jnp.int32, (s, _CBLK, n), 1) + (c_base + i * _CBLK)
    out_ref[...] = (idxt[:, None, :] == c).astype(jnp.float32)


def _part(idxt, c_base, width):
    s, n = idxt.shape
    return pl.pallas_call(
        functools.partial(_body, c_base),
        grid=(pl.cdiv(width, _CBLK),),
        in_specs=[pl.BlockSpec((s, n), lambda i: (0, 0))],
        out_specs=pl.BlockSpec((s, _CBLK, n), lambda i: (0, i, 0)),
        out_shape=jax.ShapeDtypeStruct((s, width, n), jnp.float32),
    )(idxt)


def kernel(indices):
    idxt = indices.astype(jnp.int32).T
    a = _part(idxt, 0, _SPLIT)
    b = _part(idxt, _SPLIT, _DEPTH - _SPLIT)
    out = jnp.concatenate([a, b], axis=1)
    return out.transpose(2, 0, 1)
